# Initial kernel scaffold; baseline (speedup 1.0000x reference)
import jax
import jax.numpy as jnp
from jax.experimental import pallas as pl


def kernel(x, edge_index, batch, W1_0, b1_0, W2_0, b2_0, W1_1, b1_1, W2_1, b2_1, Wf, bf):
    raise NotImplementedError("write your pallas kernel here")


# speedup vs baseline: 6.0612x; 6.0612x over previous
import functools

import jax
import jax.numpy as jnp
from jax import lax
from jax.experimental import pallas as pl
from jax.experimental.pallas import tpu as pltpu
from jax.experimental.pallas import tpu_sc as plsc

_N = 10000
_E = 320000
_D = 128
_G = 64

_NC = 2
_NS = 16
_CH = 128
_CHUNKS = _E // _CH
_CHUNKS_PER_CORE = _CHUNKS // _NC
_ITERS = -(-_CHUNKS_PER_CORE // _NS)
_ZR = 200
_ZBLOCKS = _N // _ZR
_ZITERS = -(-_ZBLOCKS // _NS)

_BM = 2000
_NB = _N // _BM


def _sc_segment_sum(x, src, dst):
    mesh = plsc.VectorSubcoreMesh(core_axis_name="core", subcore_axis_name="subcore")

    @functools.partial(
        pl.kernel,
        out_type=jax.ShapeDtypeStruct((_NC, _N, _D), jnp.float32),
        mesh=mesh,
        scratch_types=[
            pltpu.VMEM((_CH,), jnp.int32),
            pltpu.VMEM((_CH,), jnp.int32),
            pltpu.VMEM((_CH, _D), jnp.float32),
            pltpu.VMEM((_ZR, _D), jnp.float32),
            pltpu.VMEM_SHARED((_N, _D), jnp.float32),
            pltpu.SemaphoreType.DMA,
        ],
    )
    def agg(x_hbm, src_hbm, dst_hbm, out_hbm, src_v, dst_v, rows_v, zero_v,
            acc_sh, sem):
        cid = lax.axis_index("core")
        sid = lax.axis_index("subcore")

        @pl.loop(0, _ZR)
        def _(r):
            @pl.loop(0, _D, step=16)
            def _(c):
                zero_v[r, pl.ds(c, 16)] = jnp.zeros((16,), jnp.float32)

        @pl.loop(0, _ZITERS)
        def _(j):
            blk = j * _NS + sid

            @pl.when(blk < _ZBLOCKS)
            def _():
                pltpu.sync_copy(zero_v, acc_sh.at[pl.ds(blk * _ZR, _ZR)])

        plsc.subcore_barrier()

        @pl.loop(0, _ITERS)
        def _(it):
            k = it * _NS + sid

            @pl.when(k < _CHUNKS_PER_CORE)
            def _():
                base = (cid * _CHUNKS_PER_CORE + k) * _CH
                pltpu.sync_copy(src_hbm.at[pl.ds(base, _CH)], src_v)
                pltpu.sync_copy(dst_hbm.at[pl.ds(base, _CH)], dst_v)
                pltpu.async_copy(x_hbm.at[src_v], rows_v, sem).wait()
                pltpu.sync_copy(rows_v, acc_sh.at[dst_v], add=True)

        plsc.subcore_barrier()

        @pl.loop(0, _ZITERS)
        def _(j):
            blk = j * _NS + sid

            @pl.when(blk < _ZBLOCKS)
            def _():
                pltpu.sync_copy(acc_sh.at[pl.ds(blk * _ZR, _ZR)],
                                out_hbm.at[cid, pl.ds(blk * _ZR, _ZR)])

    return agg(x, src, dst)


def _mlp(x, agg, W1, b1, W2, b2):
    def body(x_ref, a0_ref, a1_ref, w1_ref, b1_ref, w2_ref, b2_ref, o_ref):
        h = x_ref[...] + a0_ref[0] + a1_ref[0]
        h = jnp.dot(h, w1_ref[...], preferred_element_type=jnp.float32)
        h = jnp.maximum(h + b1_ref[...], 0.0)
        h = jnp.dot(h, w2_ref[...], preferred_element_type=jnp.float32)
        o_ref[...] = jnp.maximum(h + b2_ref[...], 0.0)

    return pl.pallas_call(
        body,
        grid=(_NB,),
        in_specs=[
            pl.BlockSpec((_BM, _D), lambda i: (i, 0)),
            pl.BlockSpec((1, _BM, _D), lambda i: (0, i, 0)),
            pl.BlockSpec((1, _BM, _D), lambda i: (1, i, 0)),
            pl.BlockSpec((_D, _D), lambda i: (0, 0)),
            pl.BlockSpec((1, _D), lambda i: (0, 0)),
            pl.BlockSpec((_D, _D), lambda i: (0, 0)),
            pl.BlockSpec((1, _D), lambda i: (0, 0)),
        ],
        out_specs=pl.BlockSpec((_BM, _D), lambda i: (i, 0)),
        out_shape=jax.ShapeDtypeStruct((_N, _D), jnp.float32),
    )(x, agg, agg, W1, b1.reshape(1, _D), W2, b2.reshape(1, _D))


def _mlp_pool(h, agg, W1, b1, W2, b2, batch3, Wf, bf):
    def body(h_ref, a0_ref, a1_ref, w1_ref, b1_ref, w2_ref, b2_ref,
             batch_ref, wf_ref, bf_ref, o_ref, sums, counts):
        i = pl.program_id(0)

        @pl.when(i == 0)
        def _():
            sums[...] = jnp.zeros_like(sums)
            counts[...] = jnp.zeros_like(counts)

        h2 = h_ref[...] + a0_ref[0] + a1_ref[0]
        h2 = jnp.dot(h2, w1_ref[...], preferred_element_type=jnp.float32)
        h2 = jnp.maximum(h2 + b1_ref[...], 0.0)
        h2 = jnp.dot(h2, w2_ref[...], preferred_element_type=jnp.float32)
        h2 = jnp.maximum(h2 + b2_ref[...], 0.0)

        b = batch_ref[0, 0, :]
        gid = lax.broadcasted_iota(jnp.int32, (_G, _BM), 0)
        mask = (b[None, :] == gid).astype(jnp.float32)
        sums[...] += jnp.dot(mask, h2, preferred_element_type=jnp.float32)
        counts[...] += jnp.sum(mask, axis=1, keepdims=True)

        @pl.when(i == _NB - 1)
        def _():
            pooled = sums[...] / jnp.maximum(counts[...], 1.0)
            o_ref[...] = (
                jnp.dot(pooled, wf_ref[...], preferred_element_type=jnp.float32)
                + bf_ref[...])

    return pl.pallas_call(
        body,
        grid=(_NB,),
        in_specs=[
            pl.BlockSpec((_BM, _D), lambda i: (i, 0)),
            pl.BlockSpec((1, _BM, _D), lambda i: (0, i, 0)),
            pl.BlockSpec((1, _BM, _D), lambda i: (1, i, 0)),
            pl.BlockSpec((_D, _D), lambda i: (0, 0)),
            pl.BlockSpec((1, _D), lambda i: (0, 0)),
            pl.BlockSpec((_D, _D), lambda i: (0, 0)),
            pl.BlockSpec((1, _D), lambda i: (0, 0)),
            pl.BlockSpec((1, 1, _BM), lambda i: (i, 0, 0)),
            pl.BlockSpec((_D, 1), lambda i: (0, 0)),
            pl.BlockSpec((1, 1), lambda i: (0, 0)),
        ],
        out_specs=pl.BlockSpec((_G, 1), lambda i: (0, 0)),
        out_shape=jax.ShapeDtypeStruct((_G, 1), jnp.float32),
        scratch_shapes=[
            pltpu.VMEM((_G, _D), jnp.float32),
            pltpu.VMEM((_G, 1), jnp.float32),
        ],
    )(h, agg, agg, W1, b1.reshape(1, _D), W2, b2.reshape(1, _D),
      batch3, Wf, bf.reshape(1, 1))


def kernel(x, edge_index, batch, W1_0, b1_0, W2_0, b2_0, W1_1, b1_1,
           W2_1, b2_1, Wf, bf):
    src = edge_index[0]
    dst = edge_index[1]
    agg0 = _sc_segment_sum(x, src, dst)
    h1 = _mlp(x, agg0, W1_0, b1_0, W2_0, b2_0)
    agg1 = _sc_segment_sum(h1, src, dst)
    batch3 = batch.reshape(_NB, 1, _BM)
    out = _mlp_pool(h1, agg1, W1_1, b1_1, W2_1, b2_1, batch3, Wf, bf)
    return out[:, 0]
